# baseline (speedup 1.0000x reference)
import functools

import jax
import jax.numpy as jnp
from jax import lax
from jax.experimental import pallas as pl
from jax.experimental.pallas import tpu as pltpu
from jax.experimental.pallas import tpu_sc as plsc

VOCAB = 100000
EMB_DIM = 64
BATCH = 16384

_info = plsc.get_sparse_core_info()
_NC = _info.num_cores
_NS = _info.num_subcores
_NW = _NC * _NS
_RANGE = 3200
_PASSW = 512
_NPASS = -(-_RANGE // _PASSW)
_FETCHW = 512
_SA_MAX = ((VOCAB - _FETCHW) // 128) * 128
_TAILC = (VOCAB // 128) * 128
_TAILW = VOCAB - _TAILC
_PCAP = 128
_SLOTS = _PCAP // 16
_TRASH = BATCH
_IROWS = BATCH + _PCAP
_LCAP = BATCH // 16

_mesh = plsc.VectorSubcoreMesh(core_axis_name="c", subcore_axis_name="s")


@functools.partial(
    pl.kernel,
    mesh=_mesh,
    compiler_params=pltpu.CompilerParams(
        use_tc_tiling_on_sc=True, needs_layout_passes=False),
    out_type=jax.ShapeDtypeStruct((_IROWS, 128), jnp.float32),
    scratch_types=[
        pltpu.VMEM((BATCH,), jnp.int32),
        pltpu.VMEM((16, _LCAP), jnp.int32),
        pltpu.VMEM((16, 16), jnp.int32),
        pltpu.VMEM((_PCAP,), jnp.int32),
        pltpu.VMEM((64, _FETCHW), jnp.float32),
        pltpu.VMEM((64, _FETCHW), jnp.float32),
        pltpu.VMEM((64, _TAILW), jnp.float32),
        pltpu.VMEM((_PCAP, 128), jnp.float32),
        pltpu.SemaphoreType.DMA,
        pltpu.SemaphoreType.DMA,
    ],
)
def _gather_kernel(idx_hbm, tT_hbm, tail_hbm, out_hbm, idx_v, pos2_v, ppos2_v,
                   dsti_v, stripe0, stripe1, tail_v, stage_v, sem_s, sem_w):
    wid = lax.axis_index("s") * _NC + lax.axis_index("c")
    lo = wid * _RANGE
    hi = jnp.minimum(lo + _RANGE, VOCAB)
    iota16 = lax.iota(jnp.int32, 16)
    zeros16 = jnp.zeros((16,), jnp.int32)

    pltpu.sync_copy(idx_hbm, idx_v)
    pltpu.sync_copy(tail_hbm, tail_v)

    def scan_g(g, cnt16):
        v = idx_v[pl.ds(g * 16, 16)]
        m = (v >= lo) & (v < hi)
        plsc.store_scatter(pos2_v, [iota16, cnt16], iota16 + g * 16, mask=m)
        return cnt16 + m.astype(jnp.int32)

    cnt16 = lax.fori_loop(0, BATCH // 16, scan_g, zeros16)
    kmax = jnp.max(cnt16)

    stripes = [stripe0, stripe1]

    def fire(p):
        c0 = lo + p * _PASSW
        sa = jnp.minimum(c0, _SA_MAX)
        sa = pl.multiple_of(sa, 128)
        return sa, pltpu.async_copy(
            tT_hbm.at[:, pl.ds(sa, _FETCHW)], stripes[p % 2], sem_s)

    sa_cur, pending = fire(0)
    pend_w = jnp.int32(0)

    for p in range(_NPASS + 1):
        if p < _NPASS:
            c0 = lo + p * _PASSW
            cend = jnp.minimum(jnp.minimum(c0 + _PASSW, hi),
                               jnp.int32(_TAILC))
            stripe = stripes[p % 2]
            fw = _FETCHW
        else:
            c0 = jnp.int32(_TAILC)
            cend = hi
            stripe = tail_v
            sa_cur = jnp.int32(_TAILC)
            fw = _TAILW
        nxt = fire(p + 1) if p + 1 < _NPASS else None
        if p < _NPASS:
            pending.wait()

        def round_body(r, carry, c0=c0, cend=cend, sa=sa_cur, stripe=stripe,
                       fw=fw):
            pend_w, _t16 = carry

            def cmp_k(k, ordn16, c0=c0, cend=cend):
                k16 = jnp.full((16,), k, jnp.int32)
                pv = plsc.load_gather(pos2_v, [iota16, k16])
                val = plsc.load_gather(idx_v, [pv & (BATCH - 1)])
                m = (k16 < cnt16) & (val >= c0) & (val < cend)
                sel = m & (ordn16 >= r * _SLOTS) & \
                    (ordn16 < r * _SLOTS + _SLOTS)
                slot16 = jnp.clip(ordn16 - r * _SLOTS, 0, 15)
                plsc.store_scatter(ppos2_v, [iota16, slot16], pv, mask=sel)
                return ordn16 + m.astype(jnp.int32)

            total16 = lax.fori_loop(0, kmax, cmp_k, zeros16)
            pcnt16 = jnp.clip(total16 - r * _SLOTS, 0, _SLOTS)
            qmax = jnp.max(pcnt16)

            @pl.when(pend_w == 1)
            def _():
                pltpu.make_async_copy(
                    out_hbm.at[pl.ds(0, _PCAP)], stage_v, sem_w).wait()

            for qq in range(_SLOTS):
                dsti_v[pl.ds(qq * 16, 16)] = _TRASH + qq * 16 + iota16

            def ext_q(q, _, sa=sa, stripe=stripe, fw=fw):
                q16 = jnp.full((16,), q, jnp.int32)
                pp = plsc.load_gather(ppos2_v, [iota16, q16])
                lv = q16 < pcnt16
                val = plsc.load_gather(idx_v, [pp & (BATCH - 1)])
                vrel = jnp.clip(val - sa, 0, fw - 1)
                row16 = q * 16 + iota16
                dsti_v[pl.ds(q * 16, 16)] = jnp.where(lv, pp, _TRASH + row16)
                for d in range(EMB_DIM):
                    d16 = jnp.full((16,), d, jnp.int32)
                    vals = plsc.load_gather(stripe, [d16, vrel])
                    plsc.store_scatter(stage_v, [row16, d16], vals)
                return 0

            lax.fori_loop(0, qmax, ext_q, 0)

            @pl.when(qmax > 0)
            def _():
                pltpu.async_copy(stage_v, out_hbm.at[dsti_v], sem_w)

            return (jnp.where(qmax > 0, 1, pend_w * 0), total16)

        pend_w, total16 = round_body(jnp.int32(0), (pend_w, zeros16))
        extra = (jnp.max(total16) + _SLOTS - 1) // _SLOTS
        pend_w, _ = lax.fori_loop(1, extra, round_body, (pend_w, total16))

        if nxt is not None:
            sa_cur, pending = nxt

    @pl.when(pend_w == 1)
    def _():
        pltpu.make_async_copy(
            out_hbm.at[pl.ds(0, _PCAP)], stage_v, sem_w).wait()


def kernel(token_ids, embedding_weight):
    interm = _gather_kernel(token_ids.astype(jnp.int32), embedding_weight.T,
                            embedding_weight[_TAILC:, :].T)
    return interm[:BATCH, :EMB_DIM]
